# Initial kernel scaffold; baseline (speedup 1.0000x reference)
import jax
import jax.numpy as jnp
from jax.experimental import pallas as pl


def kernel(x, y, W_experts, b_experts, gate_W, gate_b):
    raise NotImplementedError("write your pallas kernel here")


# speedup vs baseline: 1.9702x; 1.9702x over previous
import functools

import jax
import jax.numpy as jnp
from jax import lax
from jax.experimental import pallas as pl
from jax.experimental.pallas import tpu as pltpu

B = 8192
DH = 1024
D = 2048
E = 8
BT = 512
NT = B // BT


def _moe_body(x_ref, y_ref, gw_ref, gb_ref, w_ref, b_ref, out_ref):
    e = pl.program_id(1)
    xv = x_ref[...]
    yv = y_ref[...]
    dnums = (((1,), (1,)), ((), ()))
    inpb = jnp.concatenate([xv, yv], axis=1).astype(jnp.bfloat16)
    gwb = gw_ref[...].astype(jnp.bfloat16)
    logits = (
        lax.dot_general(inpb, gwb, dnums, preferred_element_type=jnp.float32)
        + gb_ref[...]
    )
    idx8 = lax.broadcasted_iota(jnp.int32, (BT, E), 1)
    m1 = jnp.max(logits, axis=1, keepdims=True)
    i1 = jnp.min(jnp.where(logits == m1, idx8, E), axis=1, keepdims=True)
    l2 = jnp.where(idx8 == i1, -1e30, logits)
    m2 = jnp.max(l2, axis=1, keepdims=True)
    i2 = jnp.min(jnp.where(l2 == m2, idx8, E), axis=1, keepdims=True)
    g1 = 1.0 / (1.0 + jnp.exp(m2 - m1))
    g2 = 1.0 - g1
    w_tok = jnp.where(i1 == e, g1, 0.0) + jnp.where(i2 == e, g2, 0.0)

    xb = xv.astype(jnp.bfloat16)
    yb = yv.astype(jnp.bfloat16)
    wa = w_ref[0, :, :DH]
    wb = w_ref[0, :, DH:]
    acc = (
        lax.dot_general(xb, wa, dnums, preferred_element_type=jnp.float32)
        + lax.dot_general(yb, wb, dnums, preferred_element_type=jnp.float32)
    )
    contrib = w_tok * (acc + b_ref[0])

    @pl.when(e == 0)
    def _():
        out_ref[...] = contrib

    @pl.when(e > 0)
    def _():
        out_ref[...] += contrib


def kernel(x, y, W_experts, b_experts, gate_W, gate_b):
    Wb = W_experts.astype(jnp.bfloat16)
    b3 = b_experts.reshape(E, 1, D)
    gb2 = gate_b.reshape(1, E)
    out = pl.pallas_call(
        _moe_body,
        grid=(NT, E),
        in_specs=[
            pl.BlockSpec((BT, DH), lambda t, e: (t, 0)),
            pl.BlockSpec((BT, DH), lambda t, e: (t, 0)),
            pl.BlockSpec((E, D), lambda t, e: (0, 0)),
            pl.BlockSpec((1, E), lambda t, e: (0, 0)),
            pl.BlockSpec((1, D, D), lambda t, e: (e, 0, 0)),
            pl.BlockSpec((1, 1, D), lambda t, e: (e, 0, 0)),
        ],
        out_specs=pl.BlockSpec((BT, D), lambda t, e: (t, 0)),
        out_shape=jax.ShapeDtypeStruct((B, D), jnp.float32),
        compiler_params=pltpu.CompilerParams(
            dimension_semantics=("arbitrary", "arbitrary"),
        ),
    )(x, y, gate_W, gb2, Wb, b3)
    return out
